# Initial kernel scaffold; baseline (speedup 1.0000x reference)
import jax
import jax.numpy as jnp
from jax.experimental import pallas as pl


def kernel(x, router_w, routed_w1, routed_w2, shared_w1, shared_w2, step_t):
    raise NotImplementedError("write your pallas kernel here")


# speedup vs baseline: 3.2276x; 3.2276x over previous
import functools

import jax
import jax.numpy as jnp
from jax import lax
from jax.experimental import pallas as pl
from jax.experimental.pallas import tpu as pltpu

HIDDEN = 768
N_ROUTED = 16
N_SHARED = 2
TOP_K = 2
D_R = HIDDEN // 4
D_S = HIDDEN // 2
N_STEPS = 4

TM = 512


def _gelu(x):
    c = jnp.sqrt(2.0 / jnp.pi).astype(x.dtype)
    return 0.5 * x * (1.0 + jnp.tanh(c * (x + 0.044715 * (x * x * x))))


def _moe_body(x_ref, rw_ref, w1s_ref, w2s_ref, w1r_ref, w2r_ref, o_ref):
    x = x_ref[...]
    f32 = jnp.float32

    logits = jnp.dot(x, rw_ref[...], preferred_element_type=f32)
    lmax = jnp.max(logits, axis=-1, keepdims=True)
    ex = jnp.exp(logits - lmax)
    probs = ex / jnp.sum(ex, axis=-1, keepdims=True)

    col = lax.broadcasted_iota(jnp.int32, probs.shape, 1)
    big = jnp.int32(N_ROUTED)
    m1 = jnp.max(probs, axis=-1, keepdims=True)
    i1 = jnp.min(jnp.where(probs == m1, col, big), axis=-1, keepdims=True)
    masked = jnp.where(col == i1, -jnp.inf, probs)
    m2 = jnp.max(masked, axis=-1, keepdims=True)
    i2 = jnp.min(jnp.where(masked == m2, col, big), axis=-1, keepdims=True)
    denom = m1 + m2
    scale = (jnp.where(col == i1, m1, 0.0) + jnp.where(col == i2, m2, 0.0)) / denom

    h_s = _gelu(jnp.dot(x, w1s_ref[...], preferred_element_type=f32))
    out = jnp.dot(h_s, w2s_ref[...], preferred_element_type=f32)

    h_r = _gelu(jnp.dot(x, w1r_ref[...], preferred_element_type=f32))
    sel_r = lax.broadcasted_iota(jnp.int32, (N_ROUTED, N_ROUTED * D_R), 0)
    sel_c = lax.broadcasted_iota(jnp.int32, (N_ROUTED, N_ROUTED * D_R), 1)
    sel = (sel_c // D_R == sel_r).astype(f32)
    scale_cols = jnp.dot(scale, sel, preferred_element_type=f32)
    out = out + jnp.dot(h_r * scale_cols, w2r_ref[...], preferred_element_type=f32)

    o_ref[...] = out


def _moe_call(flat, rw, w1s, w2s, w1r, w2r):
    t_tokens = flat.shape[0]
    grid = (t_tokens // TM,)
    full = lambda shape: pl.BlockSpec(shape, lambda i: (0, 0))
    return pl.pallas_call(
        _moe_body,
        grid=grid,
        in_specs=[
            pl.BlockSpec((TM, HIDDEN), lambda i: (i, 0)),
            full((HIDDEN, N_ROUTED)),
            full((HIDDEN, N_SHARED * D_S)),
            full((N_SHARED * D_S, HIDDEN)),
            full((HIDDEN, N_ROUTED * D_R)),
            full((N_ROUTED * D_R, HIDDEN)),
        ],
        out_specs=pl.BlockSpec((TM, HIDDEN), lambda i: (i, 0)),
        out_shape=jax.ShapeDtypeStruct((t_tokens, HIDDEN), jnp.float32),
    )(flat, rw, w1s, w2s, w1r, w2r)


def kernel(x, router_w, routed_w1, routed_w2, shared_w1, shared_w2, step_t):
    orig_shape = x.shape
    flat = x.reshape(-1, orig_shape[-1])

    t = jnp.clip(jnp.asarray(step_t, jnp.int32), 0, N_STEPS - 1)
    rw = lax.dynamic_index_in_dim(router_w, t, axis=0, keepdims=False)

    w1s = jnp.transpose(shared_w1, (1, 0, 2)).reshape(HIDDEN, N_SHARED * D_S)
    w2s = shared_w2.reshape(N_SHARED * D_S, HIDDEN)
    w1r = jnp.transpose(routed_w1, (1, 0, 2)).reshape(HIDDEN, N_ROUTED * D_R)
    w2r = routed_w2.reshape(N_ROUTED * D_R, HIDDEN)

    out = _moe_call(flat, rw, w1s, w2s, w1r, w2r)
    return out.reshape(orig_shape)
